# baseline (speedup 1.0000x reference)
import functools

import jax
import jax.numpy as jnp
from jax import lax
from jax.experimental import pallas as pl
from jax.experimental.pallas import tpu as pltpu
from jax.experimental.pallas import tpu_sc as plsc

_N_USERS = 1_000_000
_EMB = 64
_B = 16384
_K_NEG = 4
_DECAY = 1e-4

_NC = 2
_NS = 16
_NW = _NC * _NS
_BPW = _B // _NW
_LANES = 16
_CPR = _EMB // _LANES
_GCH = 128
_NG = _BPW // _GCH


def _load_idx(src_hbm, base, idx_v, offset):
    for j in range(_NG):
        pltpu.sync_copy(src_hbm.at[pl.ds(base + j * _GCH, _GCH)], idx_v.at[j])
    if offset:
        for j in range(_NG):
            def _body(i, carry, j=j):
                s = pl.ds(i * _LANES, _LANES)
                idx_v[j, s] = idx_v[j, s] + offset
                return carry
            lax.fori_loop(0, _GCH // _LANES, _body, 0)


def _gather(table, idx_v, dst, sem):
    cps = [
        pltpu.async_copy(table.at[idx_v.at[j]], dst.at[pl.ds(j * _GCH, _GCH)], sem)
        for j in range(_NG)
    ]
    for cp in cps:
        cp.wait()


def _sc_body(table, user, pos, negt, diff_out, sq_out,
             idx_v, u_buf, c_buf, t_buf, dv_buf, sq_buf, sem):
    wid = lax.axis_index("s") * _NC + lax.axis_index("c")
    base = wid * _BPW

    _load_idx(user, base, idx_v, 0)
    _gather(table, idx_v, u_buf, sem)

    _load_idx(pos, base, idx_v, _N_USERS)
    _gather(table, idx_v, c_buf, sem)

    def _sq_up(r, acc):
        for j in range(_CPR):
            s = pl.ds(j * _LANES, _LANES)
            u = u_buf[r, s]
            p = c_buf[r, s]
            acc = acc + u * u + p * p
        return acc
    acc = lax.fori_loop(0, _BPW, _sq_up, jnp.zeros((_LANES,), jnp.float32))

    for k in range(_K_NEG):
        _load_idx(negt.at[k], base, idx_v, _N_USERS)
        _gather(table, idx_v, t_buf, sem)

        def _nk(r, a):
            for j in range(_CPR):
                s = pl.ds(j * _LANES, _LANES)
                t = t_buf[r, s]
                a = a + t * t
                c_buf[r, s] = c_buf[r, s] - 0.25 * t
            return a
        acc = lax.fori_loop(0, _BPW, _nk, acc)

    def _dot(r, carry):
        dv = jnp.zeros((_LANES,), jnp.float32)
        for j in range(_CPR):
            s = pl.ds(j * _LANES, _LANES)
            dv = dv + u_buf[r, s] * c_buf[r, s]
        dv_buf[r, :] = dv
        return carry
    lax.fori_loop(0, _BPW, _dot, 0)

    sq_buf[:] = acc
    pltpu.sync_copy(dv_buf, diff_out.at[pl.ds(base, _BPW)])
    pltpu.sync_copy(sq_buf, sq_out.at[wid])


_sc_gather = functools.partial(
    pl.kernel,
    mesh=plsc.VectorSubcoreMesh(core_axis_name="c", subcore_axis_name="s"),
    compiler_params=pltpu.CompilerParams(use_tc_tiling_on_sc=False),
    out_type=[
        jax.ShapeDtypeStruct((_B, _LANES), jnp.float32),
        jax.ShapeDtypeStruct((_NW, _LANES), jnp.float32),
    ],
    scratch_types=[
        pltpu.VMEM((_NG, _GCH), jnp.int32),
        pltpu.VMEM((_BPW, _EMB), jnp.float32),
        pltpu.VMEM((_BPW, _EMB), jnp.float32),
        pltpu.VMEM((_BPW, _EMB), jnp.float32),
        pltpu.VMEM((_BPW, _LANES), jnp.float32),
        pltpu.VMEM((_LANES,), jnp.float32),
        pltpu.SemaphoreType.DMA,
    ],
)(_sc_body)


def _finish_body(diff_ref, sq_ref, out_ref):
    s = jnp.sum(diff_ref[:], axis=1)
    ls = jnp.minimum(s, 0.0) - jnp.log1p(jnp.exp(-jnp.abs(s)))
    mf = -jnp.mean(ls)
    reg = jnp.sum(sq_ref[:])
    emb = _DECAY * reg * 0.5 / _B
    out_ref[0] = mf + emb
    out_ref[1] = mf
    out_ref[2] = emb


def kernel(all_embed, user, pos_item, neg_item):
    user = user.astype(jnp.int32)
    pos = pos_item.astype(jnp.int32)
    negt = neg_item.astype(jnp.int32).T

    diff, sq = _sc_gather(all_embed, user, pos, negt)

    out = pl.pallas_call(
        _finish_body,
        out_shape=jax.ShapeDtypeStruct((3,), jnp.float32),
        in_specs=[
            pl.BlockSpec(memory_space=pltpu.VMEM),
            pl.BlockSpec(memory_space=pltpu.VMEM),
        ],
        out_specs=pl.BlockSpec(memory_space=pltpu.SMEM),
    )(diff, sq)
    return (out[0], out[1], out[2])
